# speedup vs baseline: 4.9262x; 4.9262x over previous
import functools

import jax
import jax.numpy as jnp
from jax import lax
from jax.experimental import pallas as pl
from jax.experimental.pallas import tpu as pltpu
from jax.experimental.pallas import tpu_sc as plsc

VOCAB = 100000
EMB = 128
OUT = 1000
HIST = 50
BATCH = 4096

NUM_CORES = 2
NUM_SUBCORES = 16
NW = NUM_CORES * NUM_SUBCORES
B_PER_W = BATCH // NW
G = 2
CHUNK_ROWS = G * HIST
N_CHUNKS = B_PER_W // G
LANES = 16
NVREG = EMB // LANES


def _sc_body(table_hbm, idx_hbm, bow_hbm, idx_v, buf, outv, sem):
    wid = lax.axis_index("s") * NUM_CORES + lax.axis_index("c")
    pltpu.sync_copy(idx_hbm.at[pl.ds(wid * N_CHUNKS, N_CHUNKS)], idx_v)

    def chunk_body(c, _):
        pltpu.async_copy(table_hbm.at[idx_v.at[c]], buf, sem).wait()
        for g in range(G):
            def h_body(h, accs):
                return [accs[k] + buf[g * HIST + h, pl.ds(LANES * k, LANES)]
                        for k in range(NVREG)]
            accs = lax.fori_loop(
                0, HIST, h_body,
                [jnp.zeros((LANES,), jnp.float32) for _ in range(NVREG)],
                unroll=2)
            for k in range(NVREG):
                outv[c * G + g, pl.ds(LANES * k, LANES)] = accs[k]
        return ()

    lax.fori_loop(0, N_CHUNKS, chunk_body, ())
    pltpu.sync_copy(outv, bow_hbm.at[pl.ds(wid * B_PER_W, B_PER_W)])


@functools.partial(
    pl.kernel,
    out_type=jax.ShapeDtypeStruct((BATCH, EMB), jnp.float32),
    mesh=plsc.VectorSubcoreMesh(
        core_axis_name="c", subcore_axis_name="s",
        num_cores=NUM_CORES, num_subcores=NUM_SUBCORES),
    scratch_types=[
        pltpu.VMEM((N_CHUNKS, CHUNK_ROWS), jnp.int32),
        pltpu.VMEM((CHUNK_ROWS, EMB), jnp.float32),
        pltpu.VMEM((B_PER_W, EMB), jnp.float32),
        pltpu.SemaphoreType.DMA,
    ],
)
def _sc_pool(table_hbm, idx_hbm, bow_hbm, idx_v, buf, outv, sem):
    _sc_body(table_hbm, idx_hbm, bow_hbm, idx_v, buf, outv, sem)


BB = 512


def _mm_body(x_ref, wt_ref, b_ref, o_ref):
    o_ref[...] = (
        jnp.dot(x_ref[...], wt_ref[...], preferred_element_type=jnp.float32)
        + b_ref[...])


def _tc_linear(bow, wt, b2d):
    return pl.pallas_call(
        _mm_body,
        grid=(BATCH // BB,),
        in_specs=[
            pl.BlockSpec((BB, EMB), lambda i: (i, 0)),
            pl.BlockSpec((EMB, OUT), lambda i: (0, 0)),
            pl.BlockSpec((1, OUT), lambda i: (0, 0)),
        ],
        out_specs=pl.BlockSpec((BB, OUT), lambda i: (i, 0)),
        out_shape=jax.ShapeDtypeStruct((BATCH, OUT), jnp.float32),
    )(bow, wt, b2d)


def kernel(inputs, W_emb, W_lin, b_lin):
    idx2 = inputs.astype(jnp.int32).T.reshape(BATCH // G, G * HIST)
    bow = _sc_pool(W_emb, idx2)
    wt = W_lin.T
    b2d = b_lin.reshape(1, OUT)
    return _tc_linear(bow, wt, b2d)


# speedup vs baseline: 7.1895x; 1.4594x over previous
import functools

import jax
import jax.numpy as jnp
from jax import lax
from jax.experimental import pallas as pl
from jax.experimental.pallas import tpu as pltpu
from jax.experimental.pallas import tpu_sc as plsc

VOCAB = 100000
EMB = 128
OUT = 1000
HIST = 50
BATCH = 4096

NUM_CORES = 2
NUM_SUBCORES = 16
NW = NUM_CORES * NUM_SUBCORES
B_PER_W = BATCH // NW
G = 2
CHUNK_ROWS = G * HIST
N_CHUNKS = B_PER_W // G
LANES = 16
NVREG = EMB // LANES


def _accumulate(buf, outv, c):
    for g in range(G):
        def h_body(h, accs):
            return [accs[k] + buf[g * HIST + h, pl.ds(LANES * k, LANES)]
                    for k in range(NVREG)]
        accs = lax.fori_loop(
            0, HIST, h_body,
            [jnp.zeros((LANES,), jnp.float32) for _ in range(NVREG)],
            unroll=2)
        for k in range(NVREG):
            outv[c * G + g, pl.ds(LANES * k, LANES)] = accs[k]


def _sc_body(table_hbm, idx_hbm, bow_hbm, idx_v, buf0, buf1, outv, sem0, sem1):
    wid = lax.axis_index("s") * NUM_CORES + lax.axis_index("c")
    pltpu.sync_copy(idx_hbm.at[pl.ds(wid * N_CHUNKS, N_CHUNKS)], idx_v)
    pltpu.async_copy(table_hbm.at[idx_v.at[0]], buf0, sem0)

    def pair_body(t, _):
        c0 = 2 * t
        d1 = pltpu.async_copy(table_hbm.at[idx_v.at[c0 + 1]], buf1, sem1)
        pltpu.make_async_copy(
            table_hbm.at[idx_v.at[c0]], buf0, sem0).wait()
        _accumulate(buf0, outv, c0)

        @pl.when(t < N_CHUNKS // 2 - 1)
        def _():
            pltpu.async_copy(table_hbm.at[idx_v.at[c0 + 2]], buf0, sem0)

        d1.wait()
        _accumulate(buf1, outv, c0 + 1)
        return ()

    lax.fori_loop(0, N_CHUNKS // 2, pair_body, ())
    pltpu.sync_copy(outv, bow_hbm.at[pl.ds(wid * B_PER_W, B_PER_W)])


@functools.partial(
    pl.kernel,
    out_type=jax.ShapeDtypeStruct((BATCH, EMB), jnp.float32),
    mesh=plsc.VectorSubcoreMesh(
        core_axis_name="c", subcore_axis_name="s",
        num_cores=NUM_CORES, num_subcores=NUM_SUBCORES),
    scratch_types=[
        pltpu.VMEM((N_CHUNKS, CHUNK_ROWS), jnp.int32),
        pltpu.VMEM((CHUNK_ROWS, EMB), jnp.float32),
        pltpu.VMEM((CHUNK_ROWS, EMB), jnp.float32),
        pltpu.VMEM((B_PER_W, EMB), jnp.float32),
        pltpu.SemaphoreType.DMA,
        pltpu.SemaphoreType.DMA,
    ],
)
def _sc_pool(table_hbm, idx_hbm, bow_hbm, idx_v, buf0, buf1, outv, sem0, sem1):
    _sc_body(table_hbm, idx_hbm, bow_hbm, idx_v, buf0, buf1, outv, sem0, sem1)


BB = 512


def _mm_body(x_ref, wt_ref, b_ref, o_ref):
    o_ref[...] = (
        jnp.dot(x_ref[...], wt_ref[...], preferred_element_type=jnp.float32)
        + b_ref[...])


def _tc_linear(bow, wt, b2d):
    return pl.pallas_call(
        _mm_body,
        grid=(BATCH // BB,),
        in_specs=[
            pl.BlockSpec((BB, EMB), lambda i: (i, 0)),
            pl.BlockSpec((EMB, OUT), lambda i: (0, 0)),
            pl.BlockSpec((1, OUT), lambda i: (0, 0)),
        ],
        out_specs=pl.BlockSpec((BB, OUT), lambda i: (i, 0)),
        out_shape=jax.ShapeDtypeStruct((BATCH, OUT), jnp.float32),
    )(bow, wt, b2d)


def kernel(inputs, W_emb, W_lin, b_lin):
    idx2 = inputs.astype(jnp.int32).T.reshape(BATCH // G, G * HIST)
    bow = _sc_pool(W_emb, idx2)
    wt = W_lin.T
    b2d = b_lin.reshape(1, OUT)
    return _tc_linear(bow, wt, b2d)


# speedup vs baseline: 11.5747x; 1.6099x over previous
import functools

import jax
import jax.numpy as jnp
from jax import lax
from jax.experimental import pallas as pl
from jax.experimental.pallas import tpu as pltpu
from jax.experimental.pallas import tpu_sc as plsc

VOCAB = 100000
EMB = 128
OUT = 1000
HIST = 50
BATCH = 4096

NUM_CORES = 2
NUM_SUBCORES = 16
NW = NUM_CORES * NUM_SUBCORES
B_PER_W = BATCH // NW
LANES = 16
NVREG = EMB // LANES


def _sc_body(table_hbm, idx_hbm, bow_hbm, idx_v, outv, sem):
    wid = lax.axis_index("s") * NUM_CORES + lax.axis_index("c")
    pltpu.sync_copy(idx_hbm.at[:, pl.ds(wid * B_PER_W, B_PER_W)], idx_v)

    def zero_body(r, _):
        for k in range(NVREG):
            outv[r, pl.ds(LANES * k, LANES)] = jnp.zeros((LANES,), jnp.float32)
        return ()
    lax.fori_loop(0, B_PER_W, zero_body, ())

    def fire_body(h, _):
        pltpu.async_copy(table_hbm.at[idx_v.at[h]], outv, sem, add=True)
        return ()
    lax.fori_loop(0, HIST, fire_body, ())

    def drain_body(h, _):
        pltpu.make_async_copy(table_hbm.at[idx_v.at[h]], outv, sem).wait()
        return ()
    lax.fori_loop(0, HIST, drain_body, ())

    pltpu.sync_copy(outv, bow_hbm.at[pl.ds(wid * B_PER_W, B_PER_W)])


@functools.partial(
    pl.kernel,
    out_type=jax.ShapeDtypeStruct((BATCH, EMB), jnp.float32),
    mesh=plsc.VectorSubcoreMesh(
        core_axis_name="c", subcore_axis_name="s",
        num_cores=NUM_CORES, num_subcores=NUM_SUBCORES),
    scratch_types=[
        pltpu.VMEM((HIST, B_PER_W), jnp.int32),
        pltpu.VMEM((B_PER_W, EMB), jnp.float32),
        pltpu.SemaphoreType.DMA,
    ],
)
def _sc_pool(table_hbm, idx_hbm, bow_hbm, idx_v, outv, sem):
    _sc_body(table_hbm, idx_hbm, bow_hbm, idx_v, outv, sem)


BB = 1024


def _mm_body(x_ref, w_ref, b_ref, o_ref):
    o_ref[...] = lax.dot_general(
        w_ref[...], x_ref[...], (((1,), (1,)), ((), ())),
        preferred_element_type=jnp.float32) + b_ref[...]


def _tc_linear(bow, w, bcol):
    return pl.pallas_call(
        _mm_body,
        grid=(BATCH // BB,),
        in_specs=[
            pl.BlockSpec((BB, EMB), lambda i: (i, 0)),
            pl.BlockSpec((OUT, EMB), lambda i: (0, 0)),
            pl.BlockSpec((OUT, 1), lambda i: (0, 0)),
        ],
        out_specs=pl.BlockSpec((OUT, BB), lambda i: (0, i)),
        out_shape=jax.ShapeDtypeStruct((OUT, BATCH), jnp.float32),
    )(bow, w, bcol)


def kernel(inputs, W_emb, W_lin, b_lin):
    bow = _sc_pool(W_emb, inputs.astype(jnp.int32))
    bcol = b_lin.reshape(OUT, 1)
    return _tc_linear(bow, W_lin, bcol).T
